# baseline (speedup 1.0000x reference)
import functools

import jax
import jax.numpy as jnp
from jax import lax
from jax.experimental import pallas as pl
from jax.experimental.pallas import tpu as pltpu
from jax.experimental.pallas import tpu_sc as plsc


@functools.lru_cache(maxsize=None)
def _build_gather(n_tokens: int, hidden: int):
    info = plsc.get_sparse_core_info()
    nc, ns = info.num_cores, info.num_subcores
    rows_per_sc = n_tokens // nc
    chunk = 16
    grp = ns * chunk
    n_groups = rows_per_sc // grp
    nbuf = 3
    n_em = 4
    em_rows = grp // n_em

    mesh = plsc.VectorSubcoreMesh(core_axis_name="c", subcore_axis_name="s")

    @functools.partial(
        pl.kernel,
        mesh=mesh,
        out_type=jax.ShapeDtypeStruct((n_tokens, hidden), jnp.float32),
        scratch_types=[
            pltpu.VMEM((n_groups * chunk,), jnp.int32),
            pltpu.VMEM((2, chunk, hidden), jnp.float32),
            pltpu.VMEM_SHARED((nbuf, grp, hidden), jnp.float32),
            pltpu.SemaphoreType.DMA,
            pltpu.SemaphoreType.DMA,
            pltpu.SemaphoreType.DMA,
        ],
    )
    def gather_k(table_hbm, idx_hbm, out_hbm, idx_v, bufs, shared, gsem, hsem, osem):
        cid = lax.axis_index("c")
        sid = lax.axis_index("s")
        sc_base = cid * rows_per_sc

        def stage(g, c):
            pltpu.async_copy(
                idx_hbm.at[pl.ds(sc_base + g * grp + sid * chunk, chunk)],
                idx_v.at[pl.ds(g * chunk, chunk)],
                gsem,
            )
            return c

        lax.fori_loop(0, n_groups, stage, 0)

        def stage_wait(g, c):
            pltpu.make_async_copy(
                idx_hbm.at[pl.ds(sc_base + g * grp + sid * chunk, chunk)],
                idx_v.at[pl.ds(g * chunk, chunk)],
                gsem,
            ).wait()
            return c

        lax.fori_loop(0, n_groups, stage_wait, 0)

        def gather(g, tb):
            return pltpu.make_async_copy(
                table_hbm.at[idx_v.at[pl.ds(g * chunk, chunk)]], bufs.at[tb], gsem
            )

        def hop(tb, b):
            return pltpu.make_async_copy(
                bufs.at[tb], shared.at[b].at[pl.ds(sid * chunk, chunk)], hsem
            )

        def out_copy(g, b):
            return pltpu.make_async_copy(
                shared.at[b].at[pl.ds(sid * em_rows, em_rows)],
                out_hbm.at[pl.ds(sc_base + g * grp + sid * em_rows, em_rows)],
                osem,
            )

        gather(0, 0).start()

        def body(g, c):
            tb = lax.rem(g, 2)
            b = lax.rem(g, nbuf)
            gather(g, tb).wait()

            @pl.when(g + 1 < n_groups)
            def _():
                gather(g + 1, 1 - tb).start()

            @pl.when(g >= nbuf)
            def _():
                @pl.when(sid < n_em)
                def _():
                    out_copy(g - nbuf, b).wait()

                plsc.subcore_barrier()

            hop(tb, b).start()
            hop(tb, b).wait()
            plsc.subcore_barrier()

            @pl.when(sid < n_em)
            def _():
                out_copy(g, b).start()

            return c

        lax.fori_loop(0, n_groups, body, 0)

        @pl.when(sid < n_em)
        def _():
            def drain(g, c):
                out_copy(g, lax.rem(g, nbuf)).wait()
                return c

            lax.fori_loop(n_groups - nbuf, n_groups, drain, 0)

    return gather_k


def kernel(input_ids, embed_weight):
    bsz, seq = input_ids.shape
    vocab, hidden = embed_weight.shape
    ids = input_ids.reshape(-1).astype(jnp.int32)
    flat = _build_gather(bsz * seq, hidden)(embed_weight, ids)
    text_embeds = flat.reshape(bsz, seq, hidden)
    position_ids = jnp.broadcast_to(jnp.arange(seq, dtype=jnp.int32), (bsz, seq))
    lb_loss = jnp.zeros((1,), dtype=text_embeds.dtype)
    return (text_embeds, position_ids, lb_loss)


# speedup vs baseline: 1.0853x; 1.0853x over previous
import functools

import jax
import jax.numpy as jnp
from jax import lax
from jax.experimental import pallas as pl
from jax.experimental.pallas import tpu as pltpu
from jax.experimental.pallas import tpu_sc as plsc


@functools.lru_cache(maxsize=None)
def _build_gather(bsz: int, seq: int, hidden: int):
    info = plsc.get_sparse_core_info()
    nc, ns = info.num_cores, info.num_subcores
    nw = nc * ns
    n_tokens = bsz * seq
    rows_per_w = n_tokens // nw
    w_per_row = seq // rows_per_w
    chunk = 32
    n_chunks = rows_per_w // chunk

    mesh = plsc.VectorSubcoreMesh(core_axis_name="c", subcore_axis_name="s")

    @functools.partial(
        pl.kernel,
        mesh=mesh,
        out_type=jax.ShapeDtypeStruct((n_tokens, hidden), jnp.float32),
        scratch_types=[
            pltpu.VMEM((rows_per_w,), jnp.int32),
            pltpu.VMEM((2, chunk, hidden), jnp.float32),
            pltpu.SemaphoreType.DMA,
            pltpu.SemaphoreType.DMA,
            pltpu.SemaphoreType.DMA,
            pltpu.SemaphoreType.DMA,
        ],
    )
    def gather_k(table_hbm, idx_hbm, out_hbm, idx_v, bufs, g0, g1, s0, s1):
        wid = lax.axis_index("s") * nc + lax.axis_index("c")
        base = wid * rows_per_w
        pltpu.sync_copy(
            idx_hbm.at[wid // w_per_row].at[
                pl.ds(lax.rem(wid, w_per_row) * rows_per_w, rows_per_w)
            ],
            idx_v,
        )

        def gather(i, b, sem):
            return pltpu.make_async_copy(
                table_hbm.at[idx_v.at[pl.ds(i * chunk, chunk)]], bufs.at[b], sem
            )

        def scatter(i, b, sem):
            return pltpu.make_async_copy(
                bufs.at[b], out_hbm.at[pl.ds(base + i * chunk, chunk)], sem
            )

        n_groups = n_chunks // 2
        gather(0, 0, g0).start()

        def body(t, carry):
            i0 = 2 * t
            i1 = i0 + 1
            gather(i0, 0, g0).wait()
            scatter(i0, 0, s0).start()

            @pl.when(t > 0)
            def _():
                scatter(i1 - 2, 1, s1).wait()

            gather(i1, 1, g1).start()
            gather(i1, 1, g1).wait()
            scatter(i1, 1, s1).start()

            @pl.when(t + 1 < n_groups)
            def _():
                scatter(i0, 0, s0).wait()
                gather(i0 + 2, 0, g0).start()

            return carry

        lax.fori_loop(0, n_groups, body, 0)

        scatter(n_chunks - 2, 0, s0).wait()
        scatter(n_chunks - 1, 1, s1).wait()

    return gather_k


def kernel(input_ids, embed_weight):
    bsz, seq = input_ids.shape
    vocab, hidden = embed_weight.shape
    ids = input_ids.astype(jnp.int32)
    flat = _build_gather(bsz, seq, hidden)(embed_weight, ids)
    text_embeds = flat.reshape(bsz, seq, hidden)
    position_ids = jnp.broadcast_to(jnp.arange(seq, dtype=jnp.int32), (bsz, seq))
    lb_loss = jnp.zeros((1,), dtype=text_embeds.dtype)
    return (text_embeds, position_ids, lb_loss)
